# baseline (speedup 1.0000x reference)
import functools

import jax
import jax.numpy as jnp
from jax import lax
from jax.experimental import pallas as pl
from jax.experimental.pallas import tpu as pltpu
from jax.experimental.pallas import tpu_sc as plsc

_B, _H, _W, _D = 8, 32, 32, 256
_L = 16


def _sc_body(col_hbm, row_hbm, out_hbm, colv, rowv, chunk, sems):
    s = lax.axis_index("s")
    pltpu.sync_copy(col_hbm.at[pl.ds(0, _W)], colv)
    pltpu.sync_copy(row_hbm.at[pl.ds(s, 1)], rowv.at[pl.ds(0, 1)])
    pltpu.sync_copy(row_hbm.at[pl.ds(s + 16, 1)], rowv.at[pl.ds(1, 1)])

    def build_row(j, _):
        for q in range(2):
            for k in range(_D // _L):
                chunk[q, j, pl.ds(k * _L, _L)] = colv[j, pl.ds(k * _L, _L)]
            for k in range(_D // _L):
                chunk[q, j, pl.ds(_D + k * _L, _L)] = rowv[
                    q, pl.ds(k * _L, _L)
                ]
        return _

    lax.fori_loop(0, _W, build_row, 0)

    copies = [
        pltpu.async_copy(
            chunk.at[q], out_hbm.at[b, s + 16 * q], sems.at[2 * b + q]
        )
        for b in range(_B)
        for q in range(2)
    ]
    for c in copies:
        c.wait()


_sc_pos = functools.partial(
    pl.kernel,
    mesh=plsc.VectorSubcoreMesh(
        core_axis_name="c", subcore_axis_name="s", num_cores=1
    ),
    out_type=jax.ShapeDtypeStruct((_B, _H, _W, 2 * _D), jnp.float32),
    scratch_types=[
        pltpu.VMEM((_W, _D), jnp.float32),
        pltpu.VMEM((2, _D), jnp.float32),
        pltpu.VMEM((2, _W, 2 * _D), jnp.float32),
        pltpu.SemaphoreType.DMA((2 * _B,)),
    ],
)(_sc_body)


def kernel(x, row_embed, col_embed):
    out = _sc_pos(col_embed, row_embed)
    return jnp.transpose(out, (0, 3, 1, 2))


# speedup vs baseline: 1.2211x; 1.2211x over previous
import functools

import jax
import jax.numpy as jnp
from jax import lax
from jax.experimental import pallas as pl
from jax.experimental.pallas import tpu as pltpu
from jax.experimental.pallas import tpu_sc as plsc

_B, _H, _W, _D = 8, 32, 32, 256
_L = 16


def _sc_body(col_hbm, row_hbm, out_hbm, colv, rowv, chunk, sems):
    i = lax.axis_index("c") * 16 + lax.axis_index("s")
    pltpu.sync_copy(col_hbm.at[pl.ds(0, _W)], colv)
    pltpu.sync_copy(row_hbm.at[i], rowv)

    def build_row(j, _):
        for k in range(_D // _L):
            chunk[j, pl.ds(k * _L, _L)] = colv[j, pl.ds(k * _L, _L)]
        for k in range(_D // _L):
            chunk[j, pl.ds(_D + k * _L, _L)] = rowv[pl.ds(k * _L, _L)]
        return _

    lax.fori_loop(0, _W, build_row, 0)

    copies = [
        pltpu.async_copy(chunk, out_hbm.at[b, i], sems.at[b])
        for b in range(_B)
    ]
    for c in copies:
        c.wait()


_sc_pos = functools.partial(
    pl.kernel,
    mesh=plsc.VectorSubcoreMesh(
        core_axis_name="c", subcore_axis_name="s", num_cores=2
    ),
    out_type=jax.ShapeDtypeStruct((_B, _H, _W, 2 * _D), jnp.float32),
    scratch_types=[
        pltpu.VMEM((_W, _D), jnp.float32),
        pltpu.VMEM((_D,), jnp.float32),
        pltpu.VMEM((_W, 2 * _D), jnp.float32),
        pltpu.SemaphoreType.DMA((_B,)),
    ],
)(_sc_body)


def kernel(x, row_embed, col_embed):
    out = _sc_pos(col_embed, row_embed)
    return jnp.transpose(out, (0, 3, 1, 2))


# speedup vs baseline: 1.3154x; 1.0772x over previous
import functools

import jax
import jax.numpy as jnp
from jax import lax
from jax.experimental import pallas as pl
from jax.experimental.pallas import tpu as pltpu
from jax.experimental.pallas import tpu_sc as plsc

_B, _H, _W, _D = 8, 32, 32, 256
_L = 16


def _sc_body(col_hbm, row_hbm, out_hbm, rowv, chunk, sems):
    i = lax.axis_index("c") * 16 + lax.axis_index("s")
    pltpu.sync_copy(col_hbm.at[pl.ds(0, _W)], chunk.at[:, pl.ds(0, _D)])
    pltpu.sync_copy(row_hbm.at[i], rowv)

    def build_row(j, _):
        for k in range(_D // _L):
            chunk[j, pl.ds(_D + k * _L, _L)] = rowv[pl.ds(k * _L, _L)]
        return _

    lax.fori_loop(0, _W, build_row, 0)

    copies = [
        pltpu.async_copy(chunk, out_hbm.at[b, i], sems.at[b])
        for b in range(_B)
    ]
    for c in copies:
        c.wait()


_sc_pos = functools.partial(
    pl.kernel,
    mesh=plsc.VectorSubcoreMesh(
        core_axis_name="c", subcore_axis_name="s", num_cores=2
    ),
    out_type=jax.ShapeDtypeStruct((_B, _H, _W, 2 * _D), jnp.float32),
    scratch_types=[
        pltpu.VMEM((_D,), jnp.float32),
        pltpu.VMEM((_W, 2 * _D), jnp.float32),
        pltpu.SemaphoreType.DMA((_B,)),
    ],
)(_sc_body)


def kernel(x, row_embed, col_embed):
    out = _sc_pos(col_embed, row_embed)
    return jnp.transpose(out, (0, 3, 1, 2))
